# baseline (speedup 1.0000x reference)
import jax
import jax.numpy as jnp
from jax import lax
from jax.experimental import pallas as pl
from jax.experimental.pallas import tpu as pltpu
from jax.experimental.pallas import tpu_sc as plsc

NC = 2
NS = 16
NW = NC * NS
LANES = 16

BATCH = 4096
HIST = 50
EMBED_DIM = 64
BBLK = BATCH // NW
NBUF = 6
AHEAD = NBUF // 2


def _make_sc_body(h0, nch):
    def _sc_body(table_hbm, idx_hbm, inter_hbm, idx_v, b0, b1, b2, b3, b4, b5,
                 g0, g1, g2, g3, g4, g5, s0, s1, s2, s3, s4, s5, idx_sem):
        wid = lax.axis_index("s") * NC + lax.axis_index("c")
        col0 = wid * BBLK

        pltpu.sync_copy(idx_hbm.at[:, wid], idx_v)

        B = (b0, b1, b2, b3, b4, b5)
        GS = (g0, g1, g2, g3, g4, g5)
        SS = (s0, s1, s2, s3, s4, s5)

        def g_start(j, b):
            h = h0 + j
            pltpu.async_copy(table_hbm.at[idx_v.at[h // 8, h % 8]], B[b], GS[b])

        def g_wait(b):
            pltpu.make_async_copy(table_hbm.at[idx_v.at[0, 0]], B[b], GS[b]).wait()

        def s_start(j, b):
            pltpu.async_copy(B[b], inter_hbm.at[j, pl.ds(col0, BBLK)], SS[b])

        def s_wait(b):
            pltpu.make_async_copy(
                B[b], inter_hbm.at[0, pl.ds(0, BBLK)], SS[b]
            ).wait()

        for k in range(AHEAD):
            g_start(k, k)

        nsteady = (nch // NBUF) * NBUF

        def superstep(s, carry):
            for u in range(NBUF):
                j = s * NBUF + u
                b = u
                bn = (u + AHEAD) % NBUF
                g_wait(b)
                s_start(j, b)

                @pl.when(j >= AHEAD)
                def _():
                    s_wait(bn)

                @pl.when(j + AHEAD < nch)
                def _():
                    g_start(j + AHEAD, bn)
            return carry

        lax.fori_loop(0, nsteady // NBUF, superstep, 0)

        for j in range(nsteady, nch):
            b = j % NBUF
            bn = (j + AHEAD) % NBUF
            g_wait(b)
            s_start(j, b)
            s_wait(bn)

        for j in range(nch - AHEAD, nch):
            s_wait(j % NBUF)

    return _sc_body


def _tc_body(in_ref, scale_ref, out_ref):
    scale2 = jnp.concatenate([scale_ref[...], scale_ref[...]])
    fi = lax.broadcasted_iota(jnp.int32, (EMBED_DIM, 2 * EMBED_DIM), 0)
    bi = lax.broadcasted_iota(jnp.int32, (EMBED_DIM, 2 * EMBED_DIM), 1)
    sel = (fi == bi // 2).astype(jnp.float32)
    parity = bi % 2

    for g in range(2 * 32):
        xg = in_ref[pl.ds(EMBED_DIM * g, EMBED_DIM), :]
        xs = xg * scale2[None, :]
        r = lax.dot_general(
            xs, sel, (((0,), (0,)), ((), ())),
            preferred_element_type=jnp.float32,
        )
        og = jnp.where(parity == 0, r[0:EMBED_DIM, :], r[EMBED_DIM:, :])
        out_ref[g // 32, :, pl.ds(2 * EMBED_DIM * (g % 32), 2 * EMBED_DIM)] = og


def _tc_body_merge(in_ref, scale_ref, prev_ref, out_ref):
    _tc_body(in_ref, scale_ref, out_ref)


HA = 24
HB = HIST - HA


def _sc_call(h0, nch):
    mesh = plsc.VectorSubcoreMesh(
        core_axis_name="c", subcore_axis_name="s", num_cores=NC, num_subcores=NS
    )
    return pl.kernel(
        _make_sc_body(h0, nch),
        out_type=jax.ShapeDtypeStruct((nch, BATCH, EMBED_DIM), jnp.float32),
        mesh=mesh,
        compiler_params=pltpu.CompilerParams(use_tc_tiling_on_sc=False),
        scratch_types=[
            pltpu.VMEM((7, 8, BBLK), jnp.int32),
        ]
        + [pltpu.VMEM((BBLK, EMBED_DIM), jnp.float32) for _ in range(NBUF)]
        + [pltpu.SemaphoreType.DMA for _ in range(2 * NBUF)]
        + [pltpu.SemaphoreType.DMA],
    )


@jax.jit
def _run(table, idx3, scale):
    interA = _sc_call(0, HA)(table, idx3)
    interB = _sc_call(HA, HB)(table, idx3)
    interAf = interA.reshape(HA * BATCH * EMBED_DIM // 128, 128)
    interBf = interB.reshape(HB * BATCH * EMBED_DIM // 128, 128)

    outA = pl.pallas_call(
        _tc_body,
        out_shape=jax.ShapeDtypeStruct((HIST, EMBED_DIM, BATCH), jnp.float32),
        grid=(HA // 2,),
        in_specs=[
            pl.BlockSpec((2 * 2048, 128), lambda h: (h, 0)),
            pl.BlockSpec((EMBED_DIM,), lambda h: (0,)),
        ],
        out_specs=pl.BlockSpec((2, EMBED_DIM, BATCH), lambda h: (h, 0, 0)),
    )(interAf, scale)

    out_t = pl.pallas_call(
        _tc_body_merge,
        out_shape=jax.ShapeDtypeStruct((HIST, EMBED_DIM, BATCH), jnp.float32),
        grid=(HB // 2,),
        in_specs=[
            pl.BlockSpec((2 * 2048, 128), lambda h: (h, 0)),
            pl.BlockSpec((EMBED_DIM,), lambda h: (0,)),
            pl.BlockSpec((2, EMBED_DIM, BATCH), lambda h: (HA // 2, 0, 0)),
        ],
        out_specs=pl.BlockSpec((2, EMBED_DIM, BATCH), lambda h: (h + HA // 2, 0, 0)),
        input_output_aliases={2: 0},
    )(interBf, scale, outA)

    return out_t.transpose(2, 0, 1)


def kernel(x, embed_weight, p_adic_scale):
    xt = jnp.pad(x.astype(jnp.int32).T, ((0, 6), (0, 0)))
    idx4 = xt.reshape(7, 8, NW, BBLK).transpose(0, 2, 1, 3)
    return _run(embed_weight, idx4, p_adic_scale)


# speedup vs baseline: 1.0364x; 1.0364x over previous
import jax
import jax.numpy as jnp
from jax import lax
from jax.experimental import pallas as pl
from jax.experimental.pallas import tpu as pltpu
from jax.experimental.pallas import tpu_sc as plsc

NC = 2
NS = 16
NW = NC * NS
LANES = 16

BATCH = 4096
HIST = 50
EMBED_DIM = 64
BBLK = BATCH // NW
NBUF = 6
AHEAD = NBUF // 2


def _sc_body(table_hbm, idx_hbm, inter_hbm, idx_v, b0, b1, b2, b3, b4, b5,
             g0, g1, g2, g3, g4, g5, s0, s1, s2, s3, s4, s5, idx_sem):
    wid = lax.axis_index("s") * NC + lax.axis_index("c")
    col0 = wid * BBLK

    pltpu.sync_copy(idx_hbm.at[:, wid], idx_v)

    B = (b0, b1, b2, b3, b4, b5)
    GS = (g0, g1, g2, g3, g4, g5)
    SS = (s0, s1, s2, s3, s4, s5)

    def g_start(h, b):
        pltpu.async_copy(table_hbm.at[idx_v.at[h // 8, h % 8]], B[b], GS[b])

    def g_wait(b):
        pltpu.make_async_copy(table_hbm.at[idx_v.at[0, 0]], B[b], GS[b]).wait()

    def s_start(h, b):
        pltpu.async_copy(B[b], inter_hbm.at[h, pl.ds(col0, BBLK)], SS[b])

    def s_wait(b):
        pltpu.make_async_copy(B[b], inter_hbm.at[0, pl.ds(0, BBLK)], SS[b]).wait()

    for k in range(AHEAD):
        g_start(k, k)

    def superstep(s, carry):
        for u in range(NBUF):
            j = s * NBUF + u
            b = u
            bn = (u + AHEAD) % NBUF
            g_wait(b)
            s_start(j, b)

            @pl.when(j >= AHEAD)
            def _():
                s_wait(bn)

            @pl.when(j + AHEAD < HIST)
            def _():
                g_start(j + AHEAD, bn)
        return carry

    lax.fori_loop(0, 48 // NBUF, superstep, 0)

    for j in (48, 49):
        b = j % NBUF
        bn = (j + AHEAD) % NBUF
        g_wait(b)
        s_start(j, b)
        s_wait(bn)

    s_wait(5)
    s_wait(0)
    s_wait(1)


def _tc_body(in_ref, scale_ref, out_ref):
    scale2 = jnp.concatenate([scale_ref[...], scale_ref[...]])
    fi = lax.broadcasted_iota(jnp.int32, (EMBED_DIM, 2 * EMBED_DIM), 0)
    bi = lax.broadcasted_iota(jnp.int32, (EMBED_DIM, 2 * EMBED_DIM), 1)
    sel = (fi == bi // 2).astype(jnp.float32)
    parity = bi % 2

    for g in range(320):
        xg = in_ref[pl.ds(EMBED_DIM * g, EMBED_DIM), :]
        xs = xg * scale2[None, :]
        r = lax.dot_general(
            xs, sel, (((0,), (0,)), ((), ())),
            preferred_element_type=jnp.float32,
        )
        og = jnp.where(parity == 0, r[0:EMBED_DIM, :], r[EMBED_DIM:, :])
        out_ref[g // 32, :, pl.ds(2 * EMBED_DIM * (g % 32), 2 * EMBED_DIM)] = og


@jax.jit
def _run(table, idx3, scale):
    mesh = plsc.VectorSubcoreMesh(
        core_axis_name="c", subcore_axis_name="s", num_cores=NC, num_subcores=NS
    )
    sc = pl.kernel(
        _sc_body,
        out_type=jax.ShapeDtypeStruct((HIST, BATCH, EMBED_DIM), jnp.float32),
        mesh=mesh,
        compiler_params=pltpu.CompilerParams(use_tc_tiling_on_sc=False),
        scratch_types=[
            pltpu.VMEM((7, 8, BBLK), jnp.int32),
        ]
        + [pltpu.VMEM((BBLK, EMBED_DIM), jnp.float32) for _ in range(NBUF)]
        + [pltpu.SemaphoreType.DMA for _ in range(2 * NBUF)]
        + [pltpu.SemaphoreType.DMA],
    )
    inter = sc(table, idx3)
    interf = inter.reshape(HIST * BATCH * EMBED_DIM // 128, 128)

    out_t = pl.pallas_call(
        _tc_body,
        out_shape=jax.ShapeDtypeStruct((HIST, EMBED_DIM, BATCH), jnp.float32),
        grid=(HIST // 10,),
        in_specs=[
            pl.BlockSpec((20480, 128), lambda h: (h, 0)),
            pl.BlockSpec((EMBED_DIM,), lambda h: (0,)),
        ],
        out_specs=pl.BlockSpec((10, EMBED_DIM, BATCH), lambda h: (h, 0, 0)),
    )(interf, scale)

    return out_t.transpose(2, 0, 1)


def kernel(x, embed_weight, p_adic_scale):
    xt = jnp.pad(x.astype(jnp.int32).T, ((0, 6), (0, 0)))
    idx4 = xt.reshape(7, 8, NW, BBLK).transpose(0, 2, 1, 3)
    return _run(embed_weight, idx4, p_adic_scale)
